# Initial kernel scaffold; baseline (speedup 1.0000x reference)
import jax
import jax.numpy as jnp
from jax.experimental import pallas as pl


def kernel(x, edge_index, W_gcn, b_gcn, W_att, b_att, W_red, b_red):
    raise NotImplementedError("write your pallas kernel here")


# speedup vs baseline: 17.3044x; 17.3044x over previous
import jax
import jax.numpy as jnp
from jax import lax
from jax.experimental import pallas as pl
from jax.experimental.pallas import tpu as pltpu
from jax.experimental.pallas import tpu_sc as plsc

_F32 = jnp.float32
_NC = 2
_NS = 16
_NW = _NC * _NS
_K = 80


def _sc_mesh():
    return plsc.VectorSubcoreMesh(core_axis_name="c", subcore_axis_name="s")


def _zero_fill_1d(ref, n):
    def body(i, carry):
        ref[pl.ds(i * 16, 16)] = jnp.zeros((16,), _F32)
        return carry
    lax.fori_loop(0, n // 16, body, 0)


def _deg_kernel(npad, e, rpt):
    epw = e // _NW
    nb = epw // _K

    def body(dst_hbm, deg_out, deg_sh, idx_v, ones_v, lin_v):
        c = lax.axis_index("c")
        s = lax.axis_index("s")
        wid = s * _NC + c
        _zero_fill_1d(lin_v, rpt)
        def wo(i, carry):
            ones_v[pl.ds(i * 16, 16)] = jnp.ones((16,), _F32)
            return carry
        lax.fori_loop(0, _K // 16, wo, 0)
        pltpu.sync_copy(lin_v, deg_sh.at[pl.ds(s * rpt, rpt)])
        plsc.subcore_barrier()
        def step(b, carry):
            base = pl.multiple_of(wid * epw + b * _K, 8)
            pltpu.sync_copy(dst_hbm.at[pl.ds(base, _K)], idx_v)
            pltpu.sync_copy(ones_v, deg_sh.at[idx_v], add=True)
            return carry
        lax.fori_loop(0, nb, step, 0)
        plsc.subcore_barrier()
        pltpu.sync_copy(deg_sh.at[pl.ds(s * rpt, rpt)], lin_v)
        pltpu.sync_copy(lin_v, deg_out.at[c, pl.ds(s * rpt, rpt)])

    return pl.kernel(
        body,
        out_type=jax.ShapeDtypeStruct((_NC, npad), _F32),
        mesh=_sc_mesh(),
        scratch_types=[
            pltpu.VMEM_SHARED((npad,), _F32),
            pltpu.VMEM((_K,), jnp.int32),
            pltpu.VMEM((_K,), _F32),
            pltpu.VMEM((rpt,), _F32),
        ],
    )


def _agg_kernel(npad, d, e, rpt):
    epw = e // _NW
    nb = epw // _K
    nchunk = rpt // 128

    def body(src_hbm, dst_hbm, h2_hbm, s_out, s_sh, idx_s, idx_d, rows_v,
             chunk_v, sem):
        c = lax.axis_index("c")
        s = lax.axis_index("s")
        wid = s * _NC + c
        def zr(i, carry):
            r = i // (d // 16)
            col = i % (d // 16)
            chunk_v[r, pl.ds(col * 16, 16)] = jnp.zeros((16,), _F32)
            return carry
        lax.fori_loop(0, 128 * (d // 16), zr, 0)
        def zcp(j, carry):
            pltpu.sync_copy(chunk_v, s_sh.at[pl.ds(s * rpt + j * 128, 128)])
            return carry
        lax.fori_loop(0, nchunk, zcp, 0)
        plsc.subcore_barrier()
        def step(b, carry):
            base = pl.multiple_of(wid * epw + b * _K, 8)
            pltpu.sync_copy(src_hbm.at[pl.ds(base, _K)], idx_s)
            pltpu.sync_copy(dst_hbm.at[pl.ds(base, _K)], idx_d)
            pltpu.async_copy(h2_hbm.at[idx_s], rows_v, sem).wait()
            pltpu.sync_copy(rows_v, s_sh.at[idx_d], add=True)
            return carry
        lax.fori_loop(0, nb, step, 0)
        plsc.subcore_barrier()
        def dump(j, carry):
            off = pl.multiple_of(s * rpt + j * 128, 8)
            pltpu.sync_copy(s_sh.at[pl.ds(off, 128)], chunk_v)
            pltpu.sync_copy(chunk_v, s_out.at[c, pl.ds(off, 128)])
            return carry
        lax.fori_loop(0, nchunk, dump, 0)

    return pl.kernel(
        body,
        out_type=jax.ShapeDtypeStruct((_NC, npad, d), _F32),
        mesh=_sc_mesh(),
        scratch_types=[
            pltpu.VMEM_SHARED((npad, d), _F32),
            pltpu.VMEM((_K,), jnp.int32),
            pltpu.VMEM((_K,), jnp.int32),
            pltpu.VMEM((_K, d), _F32),
            pltpu.VMEM((128, d), _F32),
            pltpu.SemaphoreType.DMA,
        ],
    )


def _tca_body(n, r, dout, rank):
    def body(x_ref, w_ref, ba_ref, deg_ref, h2_ref, tmp_ref, vtz_ref,
             cucv_ref):
        i = pl.program_id(0)
        y = jnp.dot(x_ref[...], w_ref[...], preferred_element_type=_F32)
        deg = deg_ref[:, 0:1] + deg_ref[:, 1:2] + 1.0
        dinv = lax.rsqrt(jnp.maximum(deg, 1e-12))
        h2_ref[...] = y[:, :dout] * dinv
        tmpb = jnp.maximum(y[:, dout:] + ba_ref[...], 0.0)
        tmp_ref[...] = tmpb
        rows = lax.broadcasted_iota(jnp.int32, (r, 1), 0) + i * r
        mask = rows < n
        um = jnp.where(mask, tmpb[:, 0:rank], 0.0)
        vm = jnp.where(mask, tmpb[:, rank:2 * rank], 0.0)
        zb = tmpb[:, 2 * rank:3 * rank]
        vtz_b = lax.dot_general(vm, zb, (((0,), (0,)), ((), ())),
                                preferred_element_type=_F32)
        cucv_b = jnp.concatenate(
            [jnp.sum(um, axis=0, keepdims=True),
             jnp.sum(vm, axis=0, keepdims=True)], axis=0)
        @pl.when(i == 0)
        def _init():
            vtz_ref[...] = vtz_b
            cucv_ref[...] = cucv_b
        @pl.when(i > 0)
        def _acc():
            vtz_ref[...] += vtz_b
            cucv_ref[...] += cucv_b
    return body


def _tcc_body(n, rank):
    def body(s0_ref, s1_ref, h2_ref, tmp_ref, deg_ref, vtz_ref, cucv_ref,
             wr1_ref, wr2_ref, wr3_ref, bg_ref, br_ref, out_ref):
        deg = deg_ref[:, 0:1] + deg_ref[:, 1:2] + 1.0
        dinv = lax.rsqrt(jnp.maximum(deg, 1e-12))
        agg = dinv * (s0_ref[...] + s1_ref[...] + h2_ref[...]) + bg_ref[...]
        x_local = jnp.maximum(agg, 0.0)
        cu = cucv_ref[0:1, :]
        cv = cucv_ref[1:2, :]
        dn = float(n) / jnp.sum(cu * cv)
        m = jnp.dot(vtz_ref[...], wr2_ref[...],
                    preferred_element_type=_F32) * dn
        tmpb = tmp_ref[...]
        u = tmpb[:, 0:rank]
        t = tmpb[:, 3 * rank:]
        out_ref[...] = (
            jnp.dot(x_local, wr1_ref[...], preferred_element_type=_F32)
            + jnp.dot(u, m, preferred_element_type=_F32)
            + jnp.dot(t, wr3_ref[...], preferred_element_type=_F32)
            + br_ref[...])
    return body


def kernel(x, edge_index, W_gcn, b_gcn, W_att, b_att, W_red, b_red):
    n, d_in = x.shape
    e = edge_index.shape[1]
    dout = W_gcn.shape[1]
    fr = W_att.shape[1]
    rank = fr // 4
    npad = ((n + _NS * 128 - 1) // (_NS * 128)) * (_NS * 128)
    rpt = npad // _NS
    r = npad // 5
    g = npad // r

    src = edge_index[0]
    dst = edge_index[1]

    deg_parts = _deg_kernel(npad, e, rpt)(dst)
    deg_t = deg_parts.T

    xp = jnp.pad(x, ((0, npad - n), (0, 0)))
    wcat = jnp.concatenate([W_gcn, W_att], axis=1)
    h2, tmp, vtz, cucv = pl.pallas_call(
        _tca_body(n, r, dout, rank),
        grid=(g,),
        in_specs=[
            pl.BlockSpec((r, d_in), lambda i: (i, 0)),
            pl.BlockSpec((d_in, dout + fr), lambda i: (0, 0)),
            pl.BlockSpec((1, fr), lambda i: (0, 0)),
            pl.BlockSpec((r, _NC), lambda i: (i, 0)),
        ],
        out_specs=[
            pl.BlockSpec((r, dout), lambda i: (i, 0)),
            pl.BlockSpec((r, fr), lambda i: (i, 0)),
            pl.BlockSpec((rank, rank), lambda i: (0, 0)),
            pl.BlockSpec((2, rank), lambda i: (0, 0)),
        ],
        out_shape=[
            jax.ShapeDtypeStruct((npad, dout), _F32),
            jax.ShapeDtypeStruct((npad, fr), _F32),
            jax.ShapeDtypeStruct((rank, rank), _F32),
            jax.ShapeDtypeStruct((2, rank), _F32),
        ],
    )(xp, wcat, b_att.reshape(1, fr), deg_t)

    s_parts = _agg_kernel(npad, dout, e, rpt)(src, dst, h2)

    out = pl.pallas_call(
        _tcc_body(n, rank),
        grid=(g,),
        in_specs=[
            pl.BlockSpec((r, dout), lambda i: (i, 0)),
            pl.BlockSpec((r, dout), lambda i: (i, 0)),
            pl.BlockSpec((r, dout), lambda i: (i, 0)),
            pl.BlockSpec((r, fr), lambda i: (i, 0)),
            pl.BlockSpec((r, _NC), lambda i: (i, 0)),
            pl.BlockSpec((rank, rank), lambda i: (0, 0)),
            pl.BlockSpec((2, rank), lambda i: (0, 0)),
            pl.BlockSpec((dout, dout), lambda i: (0, 0)),
            pl.BlockSpec((rank, dout), lambda i: (0, 0)),
            pl.BlockSpec((rank, dout), lambda i: (0, 0)),
            pl.BlockSpec((1, dout), lambda i: (0, 0)),
            pl.BlockSpec((1, dout), lambda i: (0, 0)),
        ],
        out_specs=pl.BlockSpec((r, dout), lambda i: (i, 0)),
        out_shape=jax.ShapeDtypeStruct((npad, dout), _F32),
    )(s_parts[0], s_parts[1], h2, tmp, deg_t, vtz, cucv,
      W_red[:dout], W_red[dout:dout + rank], W_red[dout + rank:],
      b_gcn.reshape(1, dout), b_red.reshape(1, dout))

    return out[:n]


# speedup vs baseline: 38.0846x; 2.2009x over previous
import jax
import jax.numpy as jnp
from jax import lax
from jax.experimental import pallas as pl
from jax.experimental.pallas import tpu as pltpu
from jax.experimental.pallas import tpu_sc as plsc

_F32 = jnp.float32
_NC = 2
_NS = 16
_NW = _NC * _NS
_K = 125


def _sc_mesh():
    return plsc.VectorSubcoreMesh(core_axis_name="c", subcore_axis_name="s")


def _zero_fill_1d(ref, n):
    def body(i, carry):
        ref[pl.ds(i * 16, 16)] = jnp.zeros((16,), _F32)
        return carry
    lax.fori_loop(0, n // 16, body, 0)


def _deg_kernel(npad, e, rpt):
    epw = e // _NW
    nb = epw // _K
    dep = 8

    def body(dst2_hbm, deg_out, deg_sh, idx_v, ones_v, lin_v, ssem):
        c = lax.axis_index("c")
        s = lax.axis_index("s")
        wid = s * _NC + c
        _zero_fill_1d(lin_v, rpt)
        def wo(i, carry):
            ones_v[pl.ds(i * 16, 16)] = jnp.ones((16,), _F32)
            return carry
        lax.fori_loop(0, (_K + 15) // 16, wo, 0)
        pltpu.sync_copy(lin_v, deg_sh.at[pl.ds(s * rpt, rpt)])
        plsc.subcore_barrier()
        pltpu.sync_copy(dst2_hbm.at[pl.ds(wid * nb, nb)], idx_v)
        ones = ones_v.at[pl.ds(0, _K)]
        def group(i, carry):
            descs = [
                pltpu.async_copy(ones, deg_sh.at[idx_v.at[i * dep + j]],
                                 ssem, add=True)
                for j in range(dep)
            ]
            for dsc in descs:
                dsc.wait()
            return carry
        lax.fori_loop(0, nb // dep, group, 0)
        plsc.subcore_barrier()
        pltpu.sync_copy(deg_sh.at[pl.ds(s * rpt, rpt)], lin_v)
        pltpu.sync_copy(lin_v, deg_out.at[c, pl.ds(s * rpt, rpt)])

    return pl.kernel(
        body,
        out_type=jax.ShapeDtypeStruct((_NC, npad), _F32),
        mesh=_sc_mesh(),
        scratch_types=[
            pltpu.VMEM_SHARED((npad,), _F32),
            pltpu.VMEM((nb, _K), jnp.int32),
            pltpu.VMEM((((_K + 15) // 16) * 16,), _F32),
            pltpu.VMEM((rpt,), _F32),
            pltpu.SemaphoreType.DMA,
        ],
    )


def _agg_kernel(npad, d, e, rpt):
    epw = e // _NW
    nb = epw // _K
    hb = nb // 2
    nchunk = rpt // 80

    def body(src2_hbm, dst2_hbm, h2_hbm, s0_out, s1_out, s_sh, src_v, dst_v,
             rows0, rows1, gsem):
        c = lax.axis_index("c")
        s = lax.axis_index("s")
        wid = s * _NC + c
        def zr(i, carry):
            rr = i // (d // 16)
            col = i % (d // 16)
            rows0[rr, pl.ds(col * 16, 16)] = jnp.zeros((16,), _F32)
            return carry
        lax.fori_loop(0, 80 * (d // 16), zr, 0)
        def zcp(j, carry):
            pltpu.sync_copy(rows0.at[pl.ds(0, 80)],
                            s_sh.at[pl.ds(s * rpt + j * 80, 80)])
            return carry
        lax.fori_loop(0, nchunk, zcp, 0)
        plsc.subcore_barrier()

        def gather(b, rows):
            return pltpu.async_copy(h2_hbm.at[src_v.at[b]], rows, gsem)

        def scatter(b, rows):
            pltpu.sync_copy(rows, s_sh.at[dst_v.at[b]], add=True)

        for half in range(2):
            base = wid * nb + half * hb
            pltpu.sync_copy(src2_hbm.at[pl.ds(base, hb)], src_v)
            pltpu.sync_copy(dst2_hbm.at[pl.ds(base, hb)], dst_v)
            def pair(i, carry):
                d0 = gather(2 * i, rows0)
                d1 = gather(2 * i + 1, rows1)
                d0.wait()
                scatter(2 * i, rows0)
                d1.wait()
                scatter(2 * i + 1, rows1)
                return carry
            lax.fori_loop(0, hb // 2, pair, 0)
        plsc.subcore_barrier()

        def dump(out_ref):
            def dj(j, carry):
                off = s * rpt + j * 80
                pltpu.sync_copy(s_sh.at[pl.ds(off, 80)],
                                rows0.at[pl.ds(0, 80)])
                pltpu.sync_copy(rows0.at[pl.ds(0, 80)],
                                out_ref.at[pl.ds(off, 80)])
                return carry
            lax.fori_loop(0, nchunk, dj, 0)
        @pl.when(c == 0)
        def _():
            dump(s0_out)
        @pl.when(c == 1)
        def _():
            dump(s1_out)

    return pl.kernel(
        body,
        out_type=[jax.ShapeDtypeStruct((npad, d), _F32),
                  jax.ShapeDtypeStruct((npad, d), _F32)],
        mesh=_sc_mesh(),
        scratch_types=[
            pltpu.VMEM_SHARED((npad, d), _F32),
            pltpu.VMEM((hb, _K), jnp.int32),
            pltpu.VMEM((hb, _K), jnp.int32),
            pltpu.VMEM((_K, d), _F32),
            pltpu.VMEM((_K, d), _F32),
            pltpu.SemaphoreType.DMA,
        ],
    )


def _tca_body(n, r, dout, rank):
    def body(x_ref, w_ref, ba_ref, deg_ref, h2_ref, tmp_ref, vtz_ref,
             cucv_ref):
        i = pl.program_id(0)
        y = jnp.dot(x_ref[...], w_ref[...], preferred_element_type=_F32)
        deg = jnp.sum(deg_ref[...], axis=1, keepdims=True) + 1.0
        dinv = lax.rsqrt(jnp.maximum(deg, 1e-12))
        h2_ref[...] = y[:, :dout] * dinv
        tmpb = jnp.maximum(y[:, dout:] + ba_ref[...], 0.0)
        tmp_ref[...] = tmpb
        rows = lax.broadcasted_iota(jnp.int32, (r, 1), 0) + i * r
        mask = rows < n
        um = jnp.where(mask, tmpb[:, 0:rank], 0.0)
        vm = jnp.where(mask, tmpb[:, rank:2 * rank], 0.0)
        zb = tmpb[:, 2 * rank:3 * rank]
        vtz_b = lax.dot_general(vm, zb, (((0,), (0,)), ((), ())),
                                preferred_element_type=_F32)
        cucv_b = jnp.concatenate(
            [jnp.sum(um, axis=0, keepdims=True),
             jnp.sum(vm, axis=0, keepdims=True)], axis=0)
        @pl.when(i == 0)
        def _init():
            vtz_ref[...] = vtz_b
            cucv_ref[...] = cucv_b
        @pl.when(i > 0)
        def _acc():
            vtz_ref[...] += vtz_b
            cucv_ref[...] += cucv_b
    return body


def _tcc_body(n, rank):
    def body(s0_ref, s1_ref, h2_ref, tmp_ref, deg_ref, vtz_ref, cucv_ref,
             wr1_ref, wr2_ref, wr3_ref, bg_ref, br_ref, out_ref):
        deg = jnp.sum(deg_ref[...], axis=1, keepdims=True) + 1.0
        dinv = lax.rsqrt(jnp.maximum(deg, 1e-12))
        agg = dinv * (s0_ref[...] + s1_ref[...] + h2_ref[...]) + bg_ref[...]
        x_local = jnp.maximum(agg, 0.0)
        cu = cucv_ref[0:1, :]
        cv = cucv_ref[1:2, :]
        dn = float(n) / jnp.sum(cu * cv)
        m = jnp.dot(vtz_ref[...], wr2_ref[...],
                    preferred_element_type=_F32) * dn
        tmpb = tmp_ref[...]
        u = tmpb[:, 0:rank]
        t = tmpb[:, 3 * rank:]
        out_ref[...] = (
            jnp.dot(x_local, wr1_ref[...], preferred_element_type=_F32)
            + jnp.dot(u, m, preferred_element_type=_F32)
            + jnp.dot(t, wr3_ref[...], preferred_element_type=_F32)
            + br_ref[...])
    return body


def kernel(x, edge_index, W_gcn, b_gcn, W_att, b_att, W_red, b_red):
    n, d_in = x.shape
    e = edge_index.shape[1]
    dout = W_gcn.shape[1]
    fr = W_att.shape[1]
    rank = fr // 4
    npad = ((n + _NS * 128 - 1) // (_NS * 128)) * (_NS * 128)
    rpt = npad // _NS
    r = npad // 5
    g = npad // r

    src2 = edge_index[0].reshape(e // _K, _K)
    dst2 = edge_index[1].reshape(e // _K, _K)

    deg_parts = _deg_kernel(npad, e, rpt)(dst2)
    deg_t = deg_parts.T

    wcat = jnp.concatenate([W_gcn, W_att], axis=1)
    h2, tmp, vtz, cucv = pl.pallas_call(
        _tca_body(n, r, dout, rank),
        grid=(g,),
        in_specs=[
            pl.BlockSpec((r, d_in), lambda i: (i, 0)),
            pl.BlockSpec((d_in, dout + fr), lambda i: (0, 0)),
            pl.BlockSpec((1, fr), lambda i: (0, 0)),
            pl.BlockSpec((r, _NC), lambda i: (i, 0)),
        ],
        out_specs=[
            pl.BlockSpec((r, dout), lambda i: (i, 0)),
            pl.BlockSpec((r, fr), lambda i: (i, 0)),
            pl.BlockSpec((rank, rank), lambda i: (0, 0)),
            pl.BlockSpec((2, rank), lambda i: (0, 0)),
        ],
        out_shape=[
            jax.ShapeDtypeStruct((npad, dout), _F32),
            jax.ShapeDtypeStruct((npad, fr), _F32),
            jax.ShapeDtypeStruct((rank, rank), _F32),
            jax.ShapeDtypeStruct((2, rank), _F32),
        ],
    )(x, wcat, b_att.reshape(1, fr), deg_t)

    s0, s1 = _agg_kernel(npad, dout, e, rpt)(src2, dst2, h2)

    out = pl.pallas_call(
        _tcc_body(n, rank),
        grid=(g,),
        in_specs=[
            pl.BlockSpec((r, dout), lambda i: (i, 0)),
            pl.BlockSpec((r, dout), lambda i: (i, 0)),
            pl.BlockSpec((r, dout), lambda i: (i, 0)),
            pl.BlockSpec((r, fr), lambda i: (i, 0)),
            pl.BlockSpec((r, _NC), lambda i: (i, 0)),
            pl.BlockSpec((rank, rank), lambda i: (0, 0)),
            pl.BlockSpec((2, rank), lambda i: (0, 0)),
            pl.BlockSpec((dout, dout), lambda i: (0, 0)),
            pl.BlockSpec((rank, dout), lambda i: (0, 0)),
            pl.BlockSpec((rank, dout), lambda i: (0, 0)),
            pl.BlockSpec((1, dout), lambda i: (0, 0)),
            pl.BlockSpec((1, dout), lambda i: (0, 0)),
        ],
        out_specs=pl.BlockSpec((r, dout), lambda i: (i, 0)),
        out_shape=jax.ShapeDtypeStruct((n, dout), _F32),
    )(s0, s1, h2, tmp, deg_t, vtz, cucv,
      W_red[:dout], W_red[dout:dout + rank], W_red[dout + rank:],
      b_gcn.reshape(1, dout), b_red.reshape(1, dout))

    return out
